# Initial kernel scaffold; baseline (speedup 1.0000x reference)
import jax
import jax.numpy as jnp
from jax import lax
from jax.experimental import pallas as pl
from jax.experimental.pallas import tpu as pltpu
from jax.experimental.pallas import tpu_sc as plsc

N_NODES = 10000
N_EDGES = 160000
N_LABEL = 128
ALPHA = 0.15
DAMP = 1.0 - ALPHA
N_ITERS = 5
EPS = 1e-4

NC = 2
NS = 16
CHUNK = 128
NPAD = 10112
ROWS_PER_TILE = NPAD // NS
N_CHUNKS = 1264
CPT = N_CHUNKS // NS
E_PAD = N_CHUNKS * CHUNK
BH = N_LABEL // NC


def _body(src_h, dst_h, ew_h, msk_h, lsrc_h, ldst_h, ai_h, z2_h, zv_h,
          out_h, buf_a, buf_b, deg, src_v, dst_v, w_v, ew_v, msk_v,
          rows_v, dval_v, lsrc_v, ldst_v, ai_v, sc_v, out_v, sem):
    c = lax.axis_index("c")
    s = lax.axis_index("s")

    pltpu.sync_copy(src_h.at[pl.ds(s * CPT, CPT)], src_v)
    pltpu.sync_copy(dst_h.at[pl.ds(s * CPT, CPT)], dst_v)
    pltpu.sync_copy(ew_h.at[pl.ds(s * CPT, CPT)], ew_v)
    pltpu.sync_copy(msk_h.at[pl.ds(s * CPT, CPT)], msk_v)
    pltpu.sync_copy(lsrc_h.at[pl.ds(c, 1)], lsrc_v)
    pltpu.sync_copy(ldst_h.at[pl.ds(c, 1)], ldst_v)
    pltpu.sync_copy(ai_h, ai_v)

    base = s * ROWS_PER_TILE
    pltpu.sync_copy(zv_h, deg.at[pl.ds(base, ROWS_PER_TILE)])
    pltpu.sync_copy(z2_h, buf_a.at[pl.ds(base, ROWS_PER_TILE)])
    pltpu.sync_copy(z2_h, buf_b.at[pl.ds(base, ROWS_PER_TILE)])
    plsc.subcore_barrier()

    def deg_body(j, carry):
        pltpu.sync_copy(msk_v.at[j], deg.at[src_v.at[j]], add=True)
        return carry

    lax.fori_loop(0, CPT, deg_body, 0)
    plsc.subcore_barrier()

    def w_body(j, carry):
        pltpu.async_copy(deg.at[src_v.at[j]], dval_v, sem).wait()
        for g in range(CHUNK // 16):
            sl = pl.ds(g * 16, 16)
            d = dval_v[sl]
            e = ew_v[j, sl]
            w_v[j, sl] = DAMP * e / jnp.maximum(d, 1.0)
        return carry

    lax.fori_loop(0, CPT, w_body, 0)

    @pl.when(s == 0)
    def _():
        pltpu.sync_copy(ai_v, buf_a.at[lsrc_v.at[0]], add=True)

    plsc.subcore_barrier()

    def edge_phase(sbuf, dbuf):
        def body(j, carry):
            pltpu.async_copy(sbuf.at[src_v.at[j]], rows_v, sem).wait()

            def scale(i, carry2):
                ws = w_v[j, i]
                for g in range(BH // 16):
                    sl = pl.ds(g * 16, 16)
                    rows_v[i, sl] = rows_v[i, sl] * ws
                return carry2

            lax.fori_loop(0, CHUNK, scale, 0)
            pltpu.sync_copy(rows_v, dbuf.at[dst_v.at[j]], add=True)
            return carry

        lax.fori_loop(0, CPT, body, 0)

    bufs = (buf_a, buf_b)
    for it in range(N_ITERS):
        sbuf = bufs[it % 2]
        dbuf = bufs[(it + 1) % 2]
        if it >= 1:
            pltpu.sync_copy(z2_h, dbuf.at[pl.ds(base, ROWS_PER_TILE)])
            plsc.subcore_barrier()

        @pl.when(s == 0)
        def _():
            pltpu.sync_copy(ai_v, dbuf.at[lsrc_v.at[0]], add=True)

        edge_phase(sbuf, dbuf)
        plsc.subcore_barrier()

    fbuf = bufs[N_ITERS % 2]

    @pl.when(s == 0)
    def _():
        pltpu.async_copy(fbuf.at[ldst_v.at[0]], sc_v, sem).wait()
        for g in range(BH // 16):
            idx = lax.iota(jnp.int32, 16) + g * 16
            v = plsc.load_gather(sc_v, [idx, idx])
            v = jnp.where(v >= EPS, v, 0.0)
            out_v[pl.ds(g * 16, 16)] = v
        pltpu.sync_copy(out_v, out_h.at[pl.ds(c * BH, BH)])


@jax.jit
def _run(src2, dst2, ew2, msk2, lsrc2, ldst2, ai, z2, zv):
    mesh = plsc.VectorSubcoreMesh(core_axis_name="c", subcore_axis_name="s")
    f = pl.kernel(
        _body,
        out_type=jax.ShapeDtypeStruct((N_LABEL,), jnp.float32),
        mesh=mesh,
        scratch_types=[
            pltpu.VMEM_SHARED((NPAD, BH), jnp.float32),
            pltpu.VMEM_SHARED((NPAD, BH), jnp.float32),
            pltpu.VMEM_SHARED((NPAD,), jnp.float32),
            pltpu.VMEM((CPT, CHUNK), jnp.int32),
            pltpu.VMEM((CPT, CHUNK), jnp.int32),
            pltpu.VMEM((CPT, CHUNK), jnp.float32),
            pltpu.VMEM((CPT, CHUNK), jnp.float32),
            pltpu.VMEM((CPT, CHUNK), jnp.float32),
            pltpu.VMEM((CHUNK, BH), jnp.float32),
            pltpu.VMEM((CHUNK,), jnp.float32),
            pltpu.VMEM((1, BH), jnp.int32),
            pltpu.VMEM((1, BH), jnp.int32),
            pltpu.VMEM((BH, BH), jnp.float32),
            pltpu.VMEM((BH, BH), jnp.float32),
            pltpu.VMEM((BH,), jnp.float32),
            pltpu.SemaphoreType.DMA,
        ],
    )
    return f(src2, dst2, ew2, msk2, lsrc2, ldst2, ai, z2, zv)


def kernel(edge_index, edge_label_index, edge_weight):
    pad = E_PAD - N_EDGES
    src = edge_index[0].astype(jnp.int32)
    dst = edge_index[1].astype(jnp.int32)
    src2 = jnp.pad(src, (0, pad)).reshape(N_CHUNKS, CHUNK)
    dst2 = jnp.pad(dst, (0, pad)).reshape(N_CHUNKS, CHUNK)
    ew2 = jnp.pad(edge_weight.astype(jnp.float32), (0, pad)).reshape(
        N_CHUNKS, CHUNK)
    msk2 = jnp.pad(jnp.ones((N_EDGES,), jnp.float32), (0, pad)).reshape(
        N_CHUNKS, CHUNK)
    lsrc2 = edge_label_index[0].astype(jnp.int32).reshape(NC, BH)
    ldst2 = edge_label_index[1].astype(jnp.int32).reshape(NC, BH)
    ai = ALPHA * jnp.eye(BH, dtype=jnp.float32)
    z2 = jnp.zeros((ROWS_PER_TILE, BH), jnp.float32)
    zv = jnp.zeros((ROWS_PER_TILE,), jnp.float32)
    return _run(src2, dst2, ew2, msk2, lsrc2, ldst2, ai, z2, zv)


# speedup vs baseline: 4.2354x; 4.2354x over previous
import jax
import jax.numpy as jnp
from jax import lax
from jax.experimental import pallas as pl
from jax.experimental.pallas import tpu as pltpu
from jax.experimental.pallas import tpu_sc as plsc

N_NODES = 10000
N_EDGES = 160000
N_LABEL = 128
ALPHA = 0.15
DAMP = 1.0 - ALPHA
N_ITERS = 5
EPS = 1e-4

NC = 2
NS = 16
CHUNK = 128
NPAD = 10112
ROWS_PER_TILE = NPAD // NS
CPT = 80
N_CHUNKS = CPT * NS
E_PAD = N_CHUNKS * CHUNK
BH = N_LABEL // NC


def _body(src_h, dst_h, ew_h, lsrc_h, ldst_h,
          out_h, buf_a, buf_b, deg, src_v, dst_v, w_v,
          rows_v, dval_v, lsrc_v, ldst_v, ai_v, sc_v, out_v, zd_v, sem):
    c = lax.axis_index("c")
    s = lax.axis_index("s")
    lanes = lax.iota(jnp.int32, 16)
    base = s * ROWS_PER_TILE

    pltpu.sync_copy(src_h.at[s], src_v)
    pltpu.sync_copy(dst_h.at[s], dst_v)
    pltpu.sync_copy(lsrc_h.at[c], lsrc_v)
    pltpu.sync_copy(ldst_h.at[c], ldst_v)

    def mfill(j, carry):
        gbase = (s * CPT + j) * CHUNK
        for g in range(CHUNK // 16):
            gid = gbase + g * 16 + lanes
            w_v[j, pl.ds(g * 16, 16)] = jnp.where(gid < N_EDGES, 1.0, 0.0)
        return carry

    lax.fori_loop(0, CPT, mfill, 0)

    for i in range(BH):
        for g in range(BH // 16):
            v = (jnp.where(lanes == i - g * 16, ALPHA, 0.0)
                 if g == i // 16 else jnp.zeros((16,), jnp.float32))
            ai_v[i, pl.ds(g * 16, 16)] = v

    def zfill(i, carry):
        zd_v[pl.ds(i * 16, 16)] = jnp.zeros((16,), jnp.float32)
        return carry

    lax.fori_loop(0, 40, zfill, 0)

    def rfill(i, carry):
        for g in range(BH // 16):
            rows_v[i, pl.ds(g * 16, 16)] = jnp.zeros((16,), jnp.float32)
        return carry

    def zero_buf(buf):
        for off in (0, 128, 256, 384, 504):
            pltpu.sync_copy(rows_v, buf.at[pl.ds(base + off, CHUNK)])

    lax.fori_loop(0, CHUNK, rfill, 0)
    pltpu.sync_copy(zd_v.at[pl.ds(0, ROWS_PER_TILE)],
                    deg.at[pl.ds(base, ROWS_PER_TILE)])
    zero_buf(buf_a)
    zero_buf(buf_b)
    plsc.subcore_barrier()

    def deg_body(j, carry):
        pltpu.sync_copy(w_v.at[j], deg.at[src_v.at[j]], add=True)
        return carry

    lax.fori_loop(0, CPT, deg_body, 0)
    plsc.subcore_barrier()

    pltpu.sync_copy(ew_h.at[s], w_v)

    def w_body(j, carry):
        pltpu.async_copy(deg.at[src_v.at[j]], dval_v, sem).wait()
        for g in range(CHUNK // 16):
            sl = pl.ds(g * 16, 16)
            d = dval_v[sl]
            w_v[j, sl] = DAMP * w_v[j, sl] / jnp.maximum(d, 1.0)
        return carry

    lax.fori_loop(0, CPT, w_body, 0)

    @pl.when(s == 0)
    def _():
        pltpu.sync_copy(ai_v, buf_a.at[lsrc_v.at[0]], add=True)

    plsc.subcore_barrier()

    def edge_phase(sbuf, dbuf):
        def body(j, carry):
            pltpu.async_copy(sbuf.at[src_v.at[j]], rows_v, sem).wait()

            def scale(g, carry2):
                wvec = w_v[j, pl.ds(g * 16, 16)]
                ebase = g * 16
                for i in range(16):
                    ws = wvec[i]
                    for cc in range(BH // 16):
                        sl = pl.ds(cc * 16, 16)
                        rows_v[ebase + i, sl] = rows_v[ebase + i, sl] * ws
                return carry2

            lax.fori_loop(0, CHUNK // 16, scale, 0)
            pltpu.sync_copy(rows_v, dbuf.at[dst_v.at[j]], add=True)
            return carry

        lax.fori_loop(0, CPT, body, 0)

    bufs = (buf_a, buf_b)
    for it in range(N_ITERS):
        sbuf = bufs[it % 2]
        dbuf = bufs[(it + 1) % 2]
        if it >= 1:
            lax.fori_loop(0, CHUNK, rfill, 0)
            zero_buf(dbuf)
            plsc.subcore_barrier()

        @pl.when(s == 0)
        def _():
            pltpu.sync_copy(ai_v, dbuf.at[lsrc_v.at[0]], add=True)

        edge_phase(sbuf, dbuf)
        plsc.subcore_barrier()

    fbuf = bufs[N_ITERS % 2]

    @pl.when(s == 0)
    def _():
        pltpu.async_copy(fbuf.at[ldst_v.at[0]], sc_v, sem).wait()
        for g in range(BH // 16):
            sl = pl.ds(g * 16, 16)
            acc = jnp.zeros((16,), jnp.float32)
            for i in range(16):
                acc = acc + jnp.where(lanes == i, sc_v[g * 16 + i, sl], 0.0)
            acc = jnp.where(acc >= EPS, acc, 0.0)
            out_v[sl] = acc
        pltpu.sync_copy(out_v, out_h.at[pl.ds(c * BH, BH)])


@jax.jit
def _run(src2, dst2, ew2, lsrc2, ldst2):
    mesh = plsc.VectorSubcoreMesh(core_axis_name="c", subcore_axis_name="s")
    f = pl.kernel(
        _body,
        out_type=jax.ShapeDtypeStruct((N_LABEL,), jnp.float32),
        mesh=mesh,
        scratch_types=[
            pltpu.VMEM_SHARED((NPAD, BH), jnp.float32),
            pltpu.VMEM_SHARED((NPAD, BH), jnp.float32),
            pltpu.VMEM_SHARED((NPAD,), jnp.float32),
            pltpu.VMEM((CPT, CHUNK), jnp.int32),
            pltpu.VMEM((CPT, CHUNK), jnp.int32),
            pltpu.VMEM((CPT, CHUNK), jnp.float32),
            pltpu.VMEM((CHUNK, BH), jnp.float32),
            pltpu.VMEM((CHUNK,), jnp.float32),
            pltpu.VMEM((1, BH), jnp.int32),
            pltpu.VMEM((1, BH), jnp.int32),
            pltpu.VMEM((BH, BH), jnp.float32),
            pltpu.VMEM((BH, BH), jnp.float32),
            pltpu.VMEM((BH,), jnp.float32),
            pltpu.VMEM((640,), jnp.float32),
            pltpu.SemaphoreType.DMA,
        ],
        compiler_params=pltpu.CompilerParams(use_tc_tiling_on_sc=False),
    )
    return f(src2, dst2, ew2, lsrc2, ldst2)


def kernel(edge_index, edge_label_index, edge_weight):
    pad = E_PAD - N_EDGES
    src = edge_index[0].astype(jnp.int32)
    dst = edge_index[1].astype(jnp.int32)
    src2 = jnp.pad(src, (0, pad)).reshape(NS, CPT, CHUNK)
    dst2 = jnp.pad(dst, (0, pad)).reshape(NS, CPT, CHUNK)
    ew2 = jnp.pad(edge_weight.astype(jnp.float32), (0, pad)).reshape(
        NS, CPT, CHUNK)
    lsrc2 = edge_label_index[0].astype(jnp.int32).reshape(NC, 1, BH)
    ldst2 = edge_label_index[1].astype(jnp.int32).reshape(NC, 1, BH)
    return _run(src2, dst2, ew2, lsrc2, ldst2)


# speedup vs baseline: 5.0020x; 1.1810x over previous
import jax
import jax.numpy as jnp
from jax import lax
from jax.experimental import pallas as pl
from jax.experimental.pallas import tpu as pltpu
from jax.experimental.pallas import tpu_sc as plsc

N_NODES = 10000
N_EDGES = 160000
N_LABEL = 128
ALPHA = 0.15
DAMP = 1.0 - ALPHA
N_ITERS = 5
EPS = 1e-4

NC = 2
NS = 16
CHUNK = 128
NPAD = 10112
ROWS_PER_TILE = NPAD // NS
CPT = 80
N_CHUNKS = CPT * NS
E_PAD = N_CHUNKS * CHUNK
BH = N_LABEL // NC
ZOFFS = (0, 128, 256, 384, ROWS_PER_TILE - CHUNK)


def _body(src_h, dst_h, ew_h, lsrc_h, ldst_h,
          out_h, buf_a, buf_b, deg, src_v, dst_v, w_v,
          rows_a, rows_b, dval_v, lsrc_v, ldst_v, ai_v, out_v, sem):
    c = lax.axis_index("c")
    s = lax.axis_index("s")
    lanes = lax.iota(jnp.int32, 16)
    z16 = jnp.zeros((16,), jnp.float32)
    base = s * ROWS_PER_TILE

    pltpu.sync_copy(src_h.at[s], src_v)
    pltpu.sync_copy(dst_h.at[s], dst_v)
    pltpu.sync_copy(lsrc_h.at[c], lsrc_v)
    pltpu.sync_copy(ldst_h.at[c], ldst_v)

    def mfill(j, carry):
        gbase = (s * CPT + j) * CHUNK
        for g in range(CHUNK // 16):
            gid = gbase + g * 16 + lanes
            w_v[j, pl.ds(g * 16, 16)] = jnp.where(gid < N_EDGES, 1.0, 0.0)
        return carry

    lax.fori_loop(0, CPT, mfill, 0)

    def dfill(i, carry):
        dval_v[pl.ds(i * 16, 16)] = z16
        return carry

    lax.fori_loop(0, CHUNK // 16, dfill, 0)

    def rfill(i, carry):
        for g in range(BH // 16):
            rows_a[i, pl.ds(g * 16, 16)] = z16
        return carry

    def zero_buf(buf):
        for off in ZOFFS:
            pltpu.sync_copy(rows_a, buf.at[pl.ds(base + off, CHUNK)])

    lax.fori_loop(0, CHUNK, rfill, 0)
    for off in ZOFFS:
        pltpu.sync_copy(dval_v, deg.at[pl.ds(base + off, CHUNK)])
    zero_buf(buf_a)
    zero_buf(buf_b)
    plsc.subcore_barrier()

    def deg_body(j, carry):
        pltpu.sync_copy(w_v.at[j], deg.at[src_v.at[j]], add=True)
        return carry

    lax.fori_loop(0, CPT, deg_body, 0)
    plsc.subcore_barrier()

    pltpu.sync_copy(ew_h.at[s], w_v)

    def w_body(j, carry):
        pltpu.async_copy(deg.at[src_v.at[j]], dval_v, sem).wait()
        for g in range(CHUNK // 16):
            sl = pl.ds(g * 16, 16)
            d = dval_v[sl]
            w_v[j, sl] = DAMP * w_v[j, sl] / jnp.maximum(d, 1.0)
        return carry

    lax.fori_loop(0, CPT, w_body, 0)

    def inject(dbuf):
        for g4 in range(4):
            for i in range(16):
                for cc in range(BH // 16):
                    ai_v[i, pl.ds(cc * 16, 16)] = (
                        jnp.where(lanes == i, ALPHA, 0.0)
                        if cc == g4 else z16)
            pltpu.sync_copy(ai_v, dbuf.at[lsrc_v.at[g4]], add=True)

    @pl.when(s == 0)
    def _():
        inject(buf_a)

    plsc.subcore_barrier()

    def edge_phase(sbuf, dbuf):
        def start_gather(j, rbuf):
            pltpu.async_copy(sbuf.at[src_v.at[j]], rbuf, sem)

        def drain_gather(rbuf):
            pltpu.make_async_copy(sbuf.at[src_v.at[0]], rbuf, sem).wait()

        def scale_scatter(j, rbuf):
            def scale(g, carry2):
                wvec = w_v[j, pl.ds(g * 16, 16)]
                ebase = g * 16
                for i in range(16):
                    ws = wvec[i]
                    for cc in range(BH // 16):
                        sl = pl.ds(cc * 16, 16)
                        rbuf[ebase + i, sl] = rbuf[ebase + i, sl] * ws
                return carry2

            lax.fori_loop(0, CHUNK // 16, scale, 0)
            pltpu.sync_copy(rbuf, dbuf.at[dst_v.at[j]], add=True)

        start_gather(0, rows_a)

        def body(j2, carry):
            j0 = j2 * 2
            j1 = j0 + 1
            drain_gather(rows_a)
            start_gather(j1, rows_b)
            scale_scatter(j0, rows_a)
            drain_gather(rows_b)

            @pl.when(j1 + 1 < CPT)
            def _():
                start_gather(j1 + 1, rows_a)

            scale_scatter(j1, rows_b)
            return carry

        lax.fori_loop(0, CPT // 2, body, 0)

    bufs = (buf_a, buf_b)
    for it in range(N_ITERS):
        sbuf = bufs[it % 2]
        dbuf = bufs[(it + 1) % 2]
        if it >= 1:
            lax.fori_loop(0, CHUNK, rfill, 0)
            zero_buf(dbuf)
            plsc.subcore_barrier()

        @pl.when(s == 0)
        def _():
            inject(dbuf)

        edge_phase(sbuf, dbuf)
        plsc.subcore_barrier()

    fbuf = bufs[N_ITERS % 2]

    @pl.when(s == 0)
    def _():
        for g4 in range(4):
            pltpu.async_copy(fbuf.at[ldst_v.at[g4]], ai_v, sem).wait()
            sl = pl.ds(g4 * 16, 16)
            acc = z16
            for i in range(16):
                acc = acc + jnp.where(lanes == i, ai_v[i, sl], 0.0)
            acc = jnp.where(acc >= EPS, acc, 0.0)
            out_v[sl] = acc
        pltpu.sync_copy(out_v, out_h.at[pl.ds(c * BH, BH)])


@jax.jit
def _run(src2, dst2, ew2, lsrc2, ldst2):
    mesh = plsc.VectorSubcoreMesh(core_axis_name="c", subcore_axis_name="s")
    f = pl.kernel(
        _body,
        out_type=jax.ShapeDtypeStruct((N_LABEL,), jnp.float32),
        mesh=mesh,
        scratch_types=[
            pltpu.VMEM_SHARED((NPAD, BH), jnp.float32),
            pltpu.VMEM_SHARED((NPAD, BH), jnp.float32),
            pltpu.VMEM_SHARED((NPAD,), jnp.float32),
            pltpu.VMEM((CPT, CHUNK), jnp.int32),
            pltpu.VMEM((CPT, CHUNK), jnp.int32),
            pltpu.VMEM((CPT, CHUNK), jnp.float32),
            pltpu.VMEM((CHUNK, BH), jnp.float32),
            pltpu.VMEM((CHUNK, BH), jnp.float32),
            pltpu.VMEM((CHUNK,), jnp.float32),
            pltpu.VMEM((4, 16), jnp.int32),
            pltpu.VMEM((4, 16), jnp.int32),
            pltpu.VMEM((16, BH), jnp.float32),
            pltpu.VMEM((BH,), jnp.float32),
            pltpu.SemaphoreType.DMA,
        ],
        compiler_params=pltpu.CompilerParams(use_tc_tiling_on_sc=False),
    )
    return f(src2, dst2, ew2, lsrc2, ldst2)


def kernel(edge_index, edge_label_index, edge_weight):
    pad = E_PAD - N_EDGES
    src = edge_index[0].astype(jnp.int32)
    dst = edge_index[1].astype(jnp.int32)
    src2 = jnp.pad(src, (0, pad)).reshape(NS, CPT, CHUNK)
    dst2 = jnp.pad(dst, (0, pad)).reshape(NS, CPT, CHUNK)
    ew2 = jnp.pad(edge_weight.astype(jnp.float32), (0, pad)).reshape(
        NS, CPT, CHUNK)
    lsrc2 = edge_label_index[0].astype(jnp.int32).reshape(NC, 4, 16)
    ldst2 = edge_label_index[1].astype(jnp.int32).reshape(NC, 4, 16)
    return _run(src2, dst2, ew2, lsrc2, ldst2)
